# baseline (speedup 1.0000x reference)
import functools

import jax
import jax.numpy as jnp
from jax import lax
from jax.experimental import pallas as pl
from jax.experimental.pallas import tpu as pltpu
from jax.experimental.pallas import tpu_sc as plsc

VOCAB = 1000000
MAXLEN = 200
DIM = 64
BATCH = 4096

NC, NS = 2, 16
NW = NC * NS
ROWS = BATCH * MAXLEN
RPW = ROWS // NW
CHUNK = 40
NCHUNK = RPW // CHUNK
NBUF = 5
VPR = DIM // 16


def _body(x_hbm, tab_hbm, pos_hbm, out_hbm, idx_v, pos_v, bufs, gsems, ssems):
    wid = lax.axis_index("s") * NC + lax.axis_index("c")
    idx_row0 = wid * (RPW // CHUNK)
    out_row0 = wid * RPW

    pltpu.sync_copy(x_hbm.at[pl.ds(idx_row0, NCHUNK)], idx_v)
    pltpu.sync_copy(pos_hbm, pos_v)

    def start_gather(g, b):
        pltpu.async_copy(tab_hbm.at[idx_v.at[g]], bufs.at[b], gsems[b])

    def wait_gather(g, b):
        pltpu.make_async_copy(tab_hbm.at[idx_v.at[g]], bufs.at[b], gsems[b]).wait()

    def start_scatter(g, b):
        pltpu.async_copy(bufs.at[b], out_hbm.at[pl.ds(out_row0 + g * CHUNK, CHUNK)],
                         ssems[b])

    def wait_scatter(g, b):
        pltpu.make_async_copy(bufs.at[b],
                              out_hbm.at[pl.ds(out_row0 + g * CHUNK, CHUNK)],
                              ssems[b]).wait()

    for b in range(NBUF - 1):
        start_gather(b, b)

    @pl.loop(0, NCHUNK // NBUF)
    def _outer(o):
        g0 = o * NBUF
        for k in range(NBUF):
            g = g0 + k
            bn = (k + NBUF - 1) % NBUF
            @pl.when(g + NBUF - 1 < NCHUNK)
            def _issue():
                @pl.when(g >= 1)
                def _recycle():
                    wait_scatter(g - 1, bn)
                start_gather(g + NBUF - 1, bn)

            wait_gather(g, k)

            poff = CHUNK * (k % (MAXLEN // CHUNK))

            @pl.loop(0, CHUNK, unroll=4)
            def _addpos(r):
                for j in range(VPR):
                    plsc.addupdate(bufs.at[k, r, pl.ds(j * 16, 16)],
                                   pos_v[poff + r, pl.ds(j * 16, 16)])

            start_scatter(g, k)

    for k in range(NBUF):
        g = NCHUNK - NBUF + k
        wait_scatter(g, g % NBUF)


@jax.jit
def kernel(x, token_table, pos_table):
    xr = x.reshape(ROWS // CHUNK, CHUNK).astype(jnp.int32)
    mesh = plsc.VectorSubcoreMesh(core_axis_name="c", subcore_axis_name="s")
    fn = pl.kernel(
        _body,
        out_type=jax.ShapeDtypeStruct((ROWS, DIM), jnp.float32),
        mesh=mesh,
        compiler_params=pltpu.CompilerParams(use_tc_tiling_on_sc=False),
        scratch_types=[
            pltpu.VMEM((NCHUNK, CHUNK), jnp.int32),
            pltpu.VMEM((MAXLEN, DIM), jnp.float32),
            pltpu.VMEM((NBUF, CHUNK, DIM), jnp.float32),
            [pltpu.SemaphoreType.DMA] * NBUF,
            [pltpu.SemaphoreType.DMA] * NBUF,
        ],
    )
    out = fn(xr, token_table, pos_table)
    return out.reshape(BATCH, MAXLEN, DIM)


# speedup vs baseline: 1.0611x; 1.0611x over previous
import jax
import jax.numpy as jnp
from jax import lax
from jax.experimental import pallas as pl
from jax.experimental.pallas import tpu as pltpu
from jax.experimental.pallas import tpu_sc as plsc

VOCAB = 1000000
MAXLEN = 200
DIM = 64
BATCH = 4096

NC, NS = 2, 16
NW = NC * NS
ROWS = BATCH * MAXLEN
RPW = ROWS // NW
CHUNK = MAXLEN
NCHUNK = RPW // CHUNK
NBUF = 4
VPR = DIM // 16


def _body(x_hbm, tab_hbm, pos_hbm, out_hbm, idx_v, pos_v, bufs, gsems, ssems):
    wid = lax.axis_index("s") * NC + lax.axis_index("c")
    out_row0 = wid * RPW

    pltpu.sync_copy(x_hbm.at[pl.ds(wid * NCHUNK, NCHUNK)], idx_v)
    pltpu.sync_copy(pos_hbm, pos_v)

    def start_gather(g, b):
        pltpu.async_copy(tab_hbm.at[idx_v.at[g]], bufs.at[b], gsems[b])

    def wait_gather(g, b):
        pltpu.make_async_copy(tab_hbm.at[idx_v.at[g]], bufs.at[b], gsems[b]).wait()

    def start_scatter(g, b):
        pltpu.async_copy(bufs.at[b], out_hbm.at[pl.ds(out_row0 + g * CHUNK, CHUNK)],
                         ssems[b])

    def wait_scatter(g, b):
        pltpu.make_async_copy(bufs.at[b],
                              out_hbm.at[pl.ds(out_row0 + g * CHUNK, CHUNK)],
                              ssems[b]).wait()

    for b in range(NBUF - 1):
        start_gather(b, b)

    @pl.loop(0, NCHUNK // NBUF)
    def _outer(o):
        g0 = o * NBUF
        for k in range(NBUF):
            g = g0 + k
            bn = (k + NBUF - 1) % NBUF
            @pl.when(g + NBUF - 1 < NCHUNK)
            def _issue():
                @pl.when(g >= 1)
                def _recycle():
                    wait_scatter(g - 1, bn)
                start_gather(g + NBUF - 1, bn)

            wait_gather(g, k)

            @pl.loop(0, CHUNK, unroll=8)
            def _addpos(r):
                for j in range(VPR):
                    plsc.addupdate(bufs.at[k, r, pl.ds(j * 16, 16)],
                                   pos_v[r, pl.ds(j * 16, 16)])

            start_scatter(g, k)

    for k in range(NBUF):
        g = NCHUNK - NBUF + k
        wait_scatter(g, g % NBUF)


@jax.jit
def kernel(x, token_table, pos_table):
    xr = x.reshape(ROWS // CHUNK, CHUNK).astype(jnp.int32)
    mesh = plsc.VectorSubcoreMesh(core_axis_name="c", subcore_axis_name="s")
    fn = pl.kernel(
        _body,
        out_type=jax.ShapeDtypeStruct((ROWS, DIM), jnp.float32),
        mesh=mesh,
        compiler_params=pltpu.CompilerParams(use_tc_tiling_on_sc=False),
        scratch_types=[
            pltpu.VMEM((NCHUNK, CHUNK), jnp.int32),
            pltpu.VMEM((MAXLEN, DIM), jnp.float32),
            pltpu.VMEM((NBUF, CHUNK, DIM), jnp.float32),
            [pltpu.SemaphoreType.DMA] * NBUF,
            [pltpu.SemaphoreType.DMA] * NBUF,
        ],
    )
    out = fn(xr, token_table, pos_table)
    return out.reshape(BATCH, MAXLEN, DIM)
